# Initial kernel scaffold; baseline (speedup 1.0000x reference)
import jax
import jax.numpy as jnp
from jax.experimental import pallas as pl


def kernel(x, edge_index, batch, W_sem, b_sem, emb, gamma, beta, W1, b1, W2, b2, Wc1, bc1, Wc2, bc2):
    raise NotImplementedError("write your pallas kernel here")


# speedup vs baseline: 11.0627x; 11.0627x over previous
import functools

import jax
import jax.numpy as jnp
from jax import lax
from jax.experimental import pallas as pl
from jax.experimental.pallas import tpu as pltpu
from jax.experimental.pallas import tpu_sc as plsc

_N = 10000
_E = 640000
_D = 128
_B = 64
_BERT = 768
_NC = 2
_NT = 16
_NPAD = 10240
_RPT = _NPAD // _NT
_EPW = _E // (_NC * _NT)
_CH = 80
_NCH = _EPW // _CH
_NHALF = _NPAD // _NC
_RPTH = _NHALF // _NT
_ECH = _E // (_NT * _CH)
_GRP = 50
_AGRP = _ECH // _GRP
_DGRP = _NCH // _GRP
_WB = 64
_DWB = 2048
_BLK = 1000
_F32 = jnp.float32


def _gelu(x):
    return 0.5 * x * (1.0 + lax.erf(x * 0.7071067811865476))


def _embed_body(sem_ref, sidx_ref, wsem_ref, bsem_ref, emb_ref, gamma_ref,
                beta_ref, o_ref):
    h = jnp.dot(sem_ref[...], wsem_ref[...], preferred_element_type=_F32)
    h = h + bsem_ref[...]
    si = sidx_ref[...].astype(jnp.int32)
    h = h + jnp.where(si <= 0, emb_ref[0:1, :], emb_ref[1:2, :])
    m = jnp.mean(h, axis=-1, keepdims=True)
    v = jnp.mean((h - m) * (h - m), axis=-1, keepdims=True)
    h = (h - m) * lax.rsqrt(v + 1e-5) * gamma_ref[...] + beta_ref[...]
    o_ref[...] = _gelu(h)


def _embed(sem, sidx, w_sem, b_sem, emb, gamma, beta):
    grid = (_N // _BLK,)
    return pl.pallas_call(
        _embed_body,
        grid=grid,
        in_specs=[
            pl.BlockSpec((_BLK, _BERT), lambda i: (i, 0)),
            pl.BlockSpec((_BLK, 1), lambda i: (i, 0)),
            pl.BlockSpec((_BERT, _D), lambda i: (0, 0)),
            pl.BlockSpec((1, _D), lambda i: (0, 0)),
            pl.BlockSpec((2, _D), lambda i: (0, 0)),
            pl.BlockSpec((1, _D), lambda i: (0, 0)),
            pl.BlockSpec((1, _D), lambda i: (0, 0)),
        ],
        out_specs=pl.BlockSpec((_BLK, _D), lambda i: (i, 0)),
        out_shape=jax.ShapeDtypeStruct((_N, _D), _F32),
    )(sem, sidx, w_sem, b_sem, emb, gamma, beta)


def _prep_body(h_ref, w_ref, cnt0_ref, cnt1_ref, p_ref):
    g = lax.rsqrt(cnt0_ref[...] + cnt1_ref[...] + 1.0)
    p_ref[...] = jnp.dot(h_ref[...], w_ref[...],
                         preferred_element_type=_F32) * g


def _prep(h, w, cnt0, cnt1):
    grid = (_N // _BLK,)
    return pl.pallas_call(
        _prep_body,
        grid=grid,
        in_specs=[
            pl.BlockSpec((_BLK, _D), lambda i: (i, 0)),
            pl.BlockSpec((_D, _D), lambda i: (0, 0)),
            pl.BlockSpec((_BLK, 1), lambda i: (i, 0)),
            pl.BlockSpec((_BLK, 1), lambda i: (i, 0)),
        ],
        out_specs=pl.BlockSpec((_BLK, _D), lambda i: (i, 0)),
        out_shape=jax.ShapeDtypeStruct((_N, _D), _F32),
    )(h, w, cnt0, cnt1)


def _mid_body(agg_ref, p_ref, cnt0_ref, cnt1_ref, b1_ref,
              w2_ref, h1_ref, p2_ref):
    g = lax.rsqrt(cnt0_ref[...] + cnt1_ref[...] + 1.0)
    s = agg_ref[...] + p_ref[...]
    h1 = _gelu(s * g + b1_ref[...])
    h1_ref[...] = h1
    p2_ref[...] = jnp.dot(h1, w2_ref[...], preferred_element_type=_F32) * g


def _mid(agg, p, cnt0, cnt1, b1, w2):
    grid = (_N // _BLK,)
    full = pl.BlockSpec((_BLK, _D), lambda i: (i, 0))
    one = pl.BlockSpec((_BLK, 1), lambda i: (i, 0))
    return pl.pallas_call(
        _mid_body,
        grid=grid,
        in_specs=[
            full, full, one, one,
            pl.BlockSpec((1, _D), lambda i: (0, 0)),
            pl.BlockSpec((_D, _D), lambda i: (0, 0)),
        ],
        out_specs=[full, full],
        out_shape=[
            jax.ShapeDtypeStruct((_N, _D), _F32),
            jax.ShapeDtypeStruct((_N, _D), _F32),
        ],
    )(agg, p, cnt0, cnt1, b1, w2)


def _final_body(h1_ref, agg_ref, p_ref, cnt0_ref, cnt1_ref,
                b2_ref, batch_ref, wc1_ref, bc1_ref, wc2_ref, bc2_ref,
                o_ref, sums_scr, counts_scr):
    i = pl.program_id(0)

    @pl.when(i == 0)
    def _():
        sums_scr[...] = jnp.zeros_like(sums_scr)
        counts_scr[...] = jnp.zeros_like(counts_scr)

    g = lax.rsqrt(cnt0_ref[...] + cnt1_ref[...] + 1.0)
    s = agg_ref[...] + p_ref[...]
    h2 = h1_ref[...] + _gelu(s * g + b2_ref[...])
    onehot = (batch_ref[...] ==
              lax.broadcasted_iota(jnp.int32, (_BLK, _B), 1)).astype(_F32)
    dn = (((0,), (0,)), ((), ()))
    sums_scr[...] += lax.dot_general(onehot, h2, dn,
                                     preferred_element_type=_F32)
    counts_scr[...] += lax.dot_general(onehot, jnp.ones((_BLK, 1), _F32), dn,
                                       preferred_element_type=_F32)

    @pl.when(i == _N // _BLK - 1)
    def _():
        hg = sums_scr[...] / jnp.maximum(counts_scr[...], 1.0)
        z = _gelu(jnp.dot(hg, wc1_ref[...], preferred_element_type=_F32)
                  + bc1_ref[...])
        o_ref[...] = (jnp.dot(z, wc2_ref[...], preferred_element_type=_F32)
                      + bc2_ref[...])


def _final(h1, agg, p, cnt0, cnt1, b2, batch, wc1, bc1, wc2p, bc2p):
    grid = (_N // _BLK,)
    full = pl.BlockSpec((_BLK, _D), lambda i: (i, 0))
    one = pl.BlockSpec((_BLK, 1), lambda i: (i, 0))
    wfull = pl.BlockSpec((_D, _D), lambda i: (0, 0))
    brow = pl.BlockSpec((1, _D), lambda i: (0, 0))
    return pl.pallas_call(
        _final_body,
        grid=grid,
        in_specs=[full, full, full, one, one, brow,
                  pl.BlockSpec((_BLK, 1), lambda i: (i, 0)),
                  wfull, brow, wfull, brow],
        out_specs=pl.BlockSpec((_B, _D), lambda i: (0, 0)),
        out_shape=jax.ShapeDtypeStruct((_B, _D), _F32),
        scratch_shapes=[
            pltpu.VMEM((_B, _D), _F32),
            pltpu.VMEM((_B, 1), _F32),
        ],
        compiler_params=pltpu.CompilerParams(
            dimension_semantics=("arbitrary",)),
    )(h1, agg, p, cnt0, cnt1, b2, batch, wc1, bc1, wc2p, bc2p)


def _deg_sc(dst4):
    mesh = plsc.VectorSubcoreMesh(core_axis_name="c", subcore_axis_name="s",
                                  num_cores=_NC, num_subcores=_NT)

    @functools.partial(
        pl.kernel,
        out_type=jax.ShapeDtypeStruct((_NC, _NPAD), _F32),
        mesh=mesh,
        scratch_types=[
            pltpu.VMEM((_GRP, _CH), jnp.int32),
            pltpu.VMEM((_CH,), _F32),
            pltpu.VMEM((_DWB,), _F32),
            pltpu.VMEM_SHARED((_NPAD,), _F32),
        ],
    )
    def k(dst_hbm, cnt_hbm, didx_v, ones_v, buf_v, acc_sh):
        c = lax.axis_index("c")
        s = lax.axis_index("s")
        wid = c * _NT + s

        for kk in range(_CH // 16):
            ones_v[pl.ds(kk * 16, 16)] = jnp.ones((16,), _F32)

        @pl.when(s == 0)
        def _():
            def fill_zero(i, _):
                buf_v[pl.ds(i * 16, 16)] = jnp.zeros((16,), _F32)
                return 0
            lax.fori_loop(0, _DWB // 16, fill_zero, 0)
            for t in range(_NPAD // _DWB):
                pltpu.sync_copy(buf_v, acc_sh.at[pl.ds(t * _DWB, _DWB)])

        plsc.subcore_barrier()

        def group(gi, _):
            pltpu.sync_copy(dst_hbm.at[wid, gi], didx_v)

            def step(j, _2):
                pltpu.sync_copy(ones_v, acc_sh.at[didx_v.at[j]], add=True)
                return 0
            lax.fori_loop(0, _GRP, step, 0)
            return 0
        lax.fori_loop(0, _DGRP, group, 0)

        plsc.subcore_barrier()

        @pl.when(s == 0)
        def _():
            for t in range(_NPAD // _DWB):
                pltpu.sync_copy(acc_sh.at[pl.ds(t * _DWB, _DWB)], buf_v)
                pltpu.sync_copy(buf_v, cnt_hbm.at[c, pl.ds(t * _DWB, _DWB)])

    return k(dst4)


def _agg_sc(p, srcA, dstA):
    mesh = plsc.VectorSubcoreMesh(core_axis_name="c", subcore_axis_name="s",
                                  num_cores=_NC, num_subcores=_NT)

    @functools.partial(
        pl.kernel,
        out_type=jax.ShapeDtypeStruct((_NC, _NHALF, _D), _F32),
        mesh=mesh,
        scratch_types=[
            pltpu.VMEM((_GRP, _CH), jnp.int32),
            pltpu.VMEM((_GRP, _CH), jnp.int32),
            pltpu.VMEM((_CH, _D), _F32),
            pltpu.VMEM((_WB, _D), _F32),
            pltpu.VMEM_SHARED((_NHALF + 8, _D), _F32),
            pltpu.SemaphoreType.DMA,
        ],
    )
    def k(p_hbm, src_hbm, dst_hbm, out_hbm,
          sidx_v, didx_v, rows_v, buf_v, acc_sh, sem):
        c = lax.axis_index("c")
        s = lax.axis_index("s")

        def fill_zero(i, _):
            for jj in range(_D // 16):
                buf_v[i, pl.ds(jj * 16, 16)] = jnp.zeros((16,), _F32)
            return 0
        lax.fori_loop(0, _WB, fill_zero, 0)

        for t in range(_RPTH // _WB):
            pltpu.sync_copy(
                buf_v, acc_sh.at[pl.ds(s * _RPTH + t * _WB, _WB)])

        @pl.when(s == 0)
        def _():
            pltpu.sync_copy(buf_v.at[pl.ds(0, 8)],
                            acc_sh.at[pl.ds(_NHALF, 8)])

        plsc.subcore_barrier()

        base = c * _NHALF

        def group(gi, _):
            pltpu.sync_copy(src_hbm.at[s, gi], sidx_v)
            pltpu.sync_copy(dst_hbm.at[s, gi], didx_v)

            def remap(i, _2):
                for kk in range(_CH // 16):
                    d16 = didx_v[i, pl.ds(kk * 16, 16)]
                    loc = d16 - base
                    ok = jnp.logical_and(loc >= 0, loc < _NHALF)
                    didx_v[i, pl.ds(kk * 16, 16)] = jnp.where(
                        ok, loc, _NHALF)
                return 0
            lax.fori_loop(0, _GRP, remap, 0)

            def step(j, _2):
                pltpu.async_copy(p_hbm.at[sidx_v.at[j]], rows_v, sem).wait()
                pltpu.sync_copy(rows_v, acc_sh.at[didx_v.at[j]], add=True)
                return 0
            lax.fori_loop(0, _GRP, step, 0)
            return 0
        lax.fori_loop(0, _AGRP, group, 0)

        plsc.subcore_barrier()
        for t in range(_RPTH // _WB):
            pltpu.sync_copy(
                acc_sh.at[pl.ds(s * _RPTH + t * _WB, _WB)], buf_v)
            pltpu.sync_copy(
                buf_v, out_hbm.at[c, pl.ds(s * _RPTH + t * _WB, _WB)])

    return k(p, srcA, dstA)


def kernel(x, edge_index, batch, W_sem, b_sem, emb, gamma, beta, W1, b1, W2,
           b2, Wc1, bc1, Wc2, bc2):
    sem_feat = x[:, :_BERT]
    sidx = x[:, _BERT:]
    src = edge_index[0]
    dst = edge_index[1]
    dst4 = dst.reshape(_NC * _NT, _DGRP, _GRP, _CH)
    srcA = src.reshape(_NT, _AGRP, _GRP, _CH)
    dstA = dst.reshape(_NT, _AGRP, _GRP, _CH)

    cnts = _deg_sc(dst4)
    cnt0 = cnts[0].reshape(_NPAD, 1)
    cnt1 = cnts[1].reshape(_NPAD, 1)

    h0 = _embed(sem_feat, sidx, W_sem, b_sem.reshape(1, _D), emb,
                gamma.reshape(1, _D), beta.reshape(1, _D))

    p1 = _prep(h0, W1, cnt0, cnt1)
    agg1 = _agg_sc(p1, srcA, dstA).reshape(_NPAD, _D)
    h1, p2 = _mid(agg1, p1, cnt0, cnt1, b1.reshape(1, _D), W2)
    agg2 = _agg_sc(p2, srcA, dstA).reshape(_NPAD, _D)

    wc2p = jnp.concatenate([Wc2, jnp.zeros((_D, _D - 2), _F32)], axis=1)
    bc2p = jnp.concatenate([bc2, jnp.zeros((_D - 2,), _F32)]).reshape(1, _D)
    outp = _final(h1, agg2, p2, cnt0, cnt1,
                  b2.reshape(1, _D), batch.reshape(_N, 1), Wc1,
                  bc1.reshape(1, _D), wc2p, bc2p)
    return outp[:, :2]


# speedup vs baseline: 16.3501x; 1.4779x over previous
import functools

import jax
import jax.numpy as jnp
from jax import lax
from jax.experimental import pallas as pl
from jax.experimental.pallas import tpu as pltpu
from jax.experimental.pallas import tpu_sc as plsc

_N = 10000
_E = 640000
_D = 128
_B = 64
_BERT = 768
_NC = 2
_NT = 16
_NPAD = 10240
_RPT = _NPAD // _NT
_EPW = _E // (_NC * _NT)
_CH = 80
_NCH = _EPW // _CH
_NHALF = _NPAD // _NC
_RPTH = _NHALF // _NT
_ECH = _E // (_NT * _CH)
_GRP = 50
_AGRP = _ECH // _GRP
_DGRP = _NCH // _GRP
_WB = 64
_DWB = 2048
_BLK = 1000
_F32 = jnp.float32


def _gelu(x):
    return 0.5 * x * (1.0 + lax.erf(x * 0.7071067811865476))


def _embed_body(sem_ref, sidx_ref, wsem_ref, bsem_ref, emb_ref, gamma_ref,
                beta_ref, o_ref):
    h = jnp.dot(sem_ref[...], wsem_ref[...], preferred_element_type=_F32)
    h = h + bsem_ref[...]
    si = sidx_ref[...].astype(jnp.int32)
    h = h + jnp.where(si <= 0, emb_ref[0:1, :], emb_ref[1:2, :])
    m = jnp.mean(h, axis=-1, keepdims=True)
    v = jnp.mean((h - m) * (h - m), axis=-1, keepdims=True)
    h = (h - m) * lax.rsqrt(v + 1e-5) * gamma_ref[...] + beta_ref[...]
    o_ref[...] = _gelu(h)


def _embed(sem, sidx, w_sem, b_sem, emb, gamma, beta):
    grid = (_N // _BLK,)
    return pl.pallas_call(
        _embed_body,
        grid=grid,
        in_specs=[
            pl.BlockSpec((_BLK, _BERT), lambda i: (i, 0)),
            pl.BlockSpec((_BLK, 1), lambda i: (i, 0)),
            pl.BlockSpec((_BERT, _D), lambda i: (0, 0)),
            pl.BlockSpec((1, _D), lambda i: (0, 0)),
            pl.BlockSpec((2, _D), lambda i: (0, 0)),
            pl.BlockSpec((1, _D), lambda i: (0, 0)),
            pl.BlockSpec((1, _D), lambda i: (0, 0)),
        ],
        out_specs=pl.BlockSpec((_BLK, _D), lambda i: (i, 0)),
        out_shape=jax.ShapeDtypeStruct((_N, _D), _F32),
    )(sem, sidx, w_sem, b_sem, emb, gamma, beta)


def _prep_body(h_ref, w_ref, cnt0_ref, cnt1_ref, p_ref):
    g = lax.rsqrt(cnt0_ref[...] + cnt1_ref[...] + 1.0)
    p_ref[...] = jnp.dot(h_ref[...], w_ref[...],
                         preferred_element_type=_F32) * g


def _prep(h, w, cnt0, cnt1):
    grid = (_N // _BLK,)
    return pl.pallas_call(
        _prep_body,
        grid=grid,
        in_specs=[
            pl.BlockSpec((_BLK, _D), lambda i: (i, 0)),
            pl.BlockSpec((_D, _D), lambda i: (0, 0)),
            pl.BlockSpec((_BLK, 1), lambda i: (i, 0)),
            pl.BlockSpec((_BLK, 1), lambda i: (i, 0)),
        ],
        out_specs=pl.BlockSpec((_BLK, _D), lambda i: (i, 0)),
        out_shape=jax.ShapeDtypeStruct((_N, _D), _F32),
    )(h, w, cnt0, cnt1)


def _mid_body(agg_ref, p_ref, cnt0_ref, cnt1_ref, b1_ref,
              w2_ref, h1_ref, p2_ref):
    g = lax.rsqrt(cnt0_ref[...] + cnt1_ref[...] + 1.0)
    s = agg_ref[...] + p_ref[...]
    h1 = _gelu(s * g + b1_ref[...])
    h1_ref[...] = h1
    p2_ref[...] = jnp.dot(h1, w2_ref[...], preferred_element_type=_F32) * g


def _mid(agg, p, cnt0, cnt1, b1, w2):
    grid = (_N // _BLK,)
    full = pl.BlockSpec((_BLK, _D), lambda i: (i, 0))
    one = pl.BlockSpec((_BLK, 1), lambda i: (i, 0))
    return pl.pallas_call(
        _mid_body,
        grid=grid,
        in_specs=[
            full, full, one, one,
            pl.BlockSpec((1, _D), lambda i: (0, 0)),
            pl.BlockSpec((_D, _D), lambda i: (0, 0)),
        ],
        out_specs=[full, full],
        out_shape=[
            jax.ShapeDtypeStruct((_N, _D), _F32),
            jax.ShapeDtypeStruct((_N, _D), _F32),
        ],
    )(agg, p, cnt0, cnt1, b1, w2)


def _final_body(h1_ref, agg_ref, p_ref, cnt0_ref, cnt1_ref,
                b2_ref, batch_ref, wc1_ref, bc1_ref, wc2_ref, bc2_ref,
                o_ref, sums_scr, counts_scr):
    i = pl.program_id(0)

    @pl.when(i == 0)
    def _():
        sums_scr[...] = jnp.zeros_like(sums_scr)
        counts_scr[...] = jnp.zeros_like(counts_scr)

    g = lax.rsqrt(cnt0_ref[...] + cnt1_ref[...] + 1.0)
    s = agg_ref[...] + p_ref[...]
    h2 = h1_ref[...] + _gelu(s * g + b2_ref[...])
    onehot = (batch_ref[...] ==
              lax.broadcasted_iota(jnp.int32, (_BLK, _B), 1)).astype(_F32)
    dn = (((0,), (0,)), ((), ()))
    sums_scr[...] += lax.dot_general(onehot, h2, dn,
                                     preferred_element_type=_F32)
    counts_scr[...] += lax.dot_general(onehot, jnp.ones((_BLK, 1), _F32), dn,
                                       preferred_element_type=_F32)

    @pl.when(i == _N // _BLK - 1)
    def _():
        hg = sums_scr[...] / jnp.maximum(counts_scr[...], 1.0)
        z = _gelu(jnp.dot(hg, wc1_ref[...], preferred_element_type=_F32)
                  + bc1_ref[...])
        o_ref[...] = (jnp.dot(z, wc2_ref[...], preferred_element_type=_F32)
                      + bc2_ref[...])


def _final(h1, agg, p, cnt0, cnt1, b2, batch, wc1, bc1, wc2p, bc2p):
    grid = (_N // _BLK,)
    full = pl.BlockSpec((_BLK, _D), lambda i: (i, 0))
    one = pl.BlockSpec((_BLK, 1), lambda i: (i, 0))
    wfull = pl.BlockSpec((_D, _D), lambda i: (0, 0))
    brow = pl.BlockSpec((1, _D), lambda i: (0, 0))
    return pl.pallas_call(
        _final_body,
        grid=grid,
        in_specs=[full, full, full, one, one, brow,
                  pl.BlockSpec((_BLK, 1), lambda i: (i, 0)),
                  wfull, brow, wfull, brow],
        out_specs=pl.BlockSpec((_B, _D), lambda i: (0, 0)),
        out_shape=jax.ShapeDtypeStruct((_B, _D), _F32),
        scratch_shapes=[
            pltpu.VMEM((_B, _D), _F32),
            pltpu.VMEM((_B, 1), _F32),
        ],
        compiler_params=pltpu.CompilerParams(
            dimension_semantics=("arbitrary",)),
    )(h1, agg, p, cnt0, cnt1, b2, batch, wc1, bc1, wc2p, bc2p)


def _deg_sc(dst4):
    mesh = plsc.VectorSubcoreMesh(core_axis_name="c", subcore_axis_name="s",
                                  num_cores=_NC, num_subcores=_NT)

    @functools.partial(
        pl.kernel,
        out_type=jax.ShapeDtypeStruct((_NC, _NPAD), _F32),
        mesh=mesh,
        scratch_types=[
            pltpu.VMEM((_GRP, _CH), jnp.int32),
            pltpu.VMEM((_CH,), _F32),
            pltpu.VMEM((_DWB,), _F32),
            pltpu.VMEM_SHARED((_NPAD,), _F32),
            pltpu.SemaphoreType.DMA,
        ],
    )
    def k(dst_hbm, cnt_hbm, didx_v, ones_v, buf_v, acc_sh, dsem):
        c = lax.axis_index("c")
        s = lax.axis_index("s")
        wid = c * _NT + s

        for kk in range(_CH // 16):
            ones_v[pl.ds(kk * 16, 16)] = jnp.ones((16,), _F32)

        @pl.when(s == 0)
        def _():
            def fill_zero(i, _):
                buf_v[pl.ds(i * 16, 16)] = jnp.zeros((16,), _F32)
                return 0
            lax.fori_loop(0, _DWB // 16, fill_zero, 0)
            for t in range(_NPAD // _DWB):
                pltpu.sync_copy(buf_v, acc_sh.at[pl.ds(t * _DWB, _DWB)])

        plsc.subcore_barrier()

        def group(gi, _):
            pltpu.sync_copy(dst_hbm.at[wid, gi], didx_v)
            for sb in range(_GRP // 10):
                ds = [pltpu.async_copy(
                    ones_v, acc_sh.at[didx_v.at[sb * 10 + t]], dsem,
                    add=True) for t in range(10)]
                for d in ds:
                    d.wait()
            return 0
        lax.fori_loop(0, _DGRP, group, 0)

        plsc.subcore_barrier()

        @pl.when(s == 0)
        def _():
            for t in range(_NPAD // _DWB):
                pltpu.sync_copy(acc_sh.at[pl.ds(t * _DWB, _DWB)], buf_v)
                pltpu.sync_copy(buf_v, cnt_hbm.at[c, pl.ds(t * _DWB, _DWB)])

    return k(dst4)


def _agg_sc(p, srcA, dstA):
    mesh = plsc.VectorSubcoreMesh(core_axis_name="c", subcore_axis_name="s",
                                  num_cores=_NC, num_subcores=_NT)

    @functools.partial(
        pl.kernel,
        out_type=jax.ShapeDtypeStruct((_NC, _NHALF, _D), _F32),
        mesh=mesh,
        scratch_types=[
            pltpu.VMEM((_GRP, _CH), jnp.int32),
            pltpu.VMEM((_GRP, _CH), jnp.int32),
            pltpu.VMEM((_CH, _D), _F32),
            pltpu.VMEM((_CH, _D), _F32),
            pltpu.VMEM((_CH, _D), _F32),
            pltpu.VMEM((_CH, _D), _F32),
            pltpu.VMEM((_WB, _D), _F32),
            pltpu.VMEM_SHARED((_NHALF + 8, _D), _F32),
            pltpu.SemaphoreType.DMA,
            pltpu.SemaphoreType.DMA,
            pltpu.SemaphoreType.DMA,
            pltpu.SemaphoreType.DMA,
            pltpu.SemaphoreType.DMA,
            pltpu.SemaphoreType.DMA,
            pltpu.SemaphoreType.DMA,
            pltpu.SemaphoreType.DMA,
        ],
    )
    def k(p_hbm, src_hbm, dst_hbm, out_hbm,
          sidx_v, didx_v, r0, r1, r2, r3, buf_v, acc_sh,
          g0, g1, g2, g3, s0, s1, s2, s3):
        c = lax.axis_index("c")
        s = lax.axis_index("s")
        rows = (r0, r1, r2, r3)
        gsem = (g0, g1, g2, g3)
        ssem = (s0, s1, s2, s3)

        def fill_zero(i, _):
            for jj in range(_D // 16):
                buf_v[i, pl.ds(jj * 16, 16)] = jnp.zeros((16,), _F32)
            return 0
        lax.fori_loop(0, _WB, fill_zero, 0)

        for t in range(_RPTH // _WB):
            pltpu.sync_copy(
                buf_v, acc_sh.at[pl.ds(s * _RPTH + t * _WB, _WB)])

        @pl.when(s == 0)
        def _():
            pltpu.sync_copy(buf_v.at[pl.ds(0, 8)],
                            acc_sh.at[pl.ds(_NHALF, 8)])

        plsc.subcore_barrier()

        base = c * _NHALF

        def gath(j, b):
            return pltpu.async_copy(p_hbm.at[sidx_v.at[j]], rows[b],
                                    gsem[b])

        def scat(j, b):
            return pltpu.async_copy(rows[b], acc_sh.at[didx_v.at[j]],
                                    ssem[b], add=True)

        def group(gi, _):
            pltpu.sync_copy(src_hbm.at[s, gi], sidx_v)
            pltpu.sync_copy(dst_hbm.at[s, gi], didx_v)

            def remap(i, _2):
                for kk in range(_CH // 16):
                    d16 = didx_v[i, pl.ds(kk * 16, 16)]
                    loc = d16 - base
                    ok = jnp.logical_and(loc >= 0, loc < _NHALF)
                    didx_v[i, pl.ds(kk * 16, 16)] = jnp.where(
                        ok, loc, _NHALF)
                return 0
            lax.fori_loop(0, _GRP, remap, 0)

            d0 = gath(0, 0)
            d1 = gath(1, 1)
            d0.wait()
            e0 = scat(0, 0)
            d2 = gath(2, 2)
            d1.wait()
            e1 = scat(1, 1)
            d3 = gath(3, 3)
            d2.wait()
            scat(2, 2)
            e0.wait()
            gath(4, 0)
            d3.wait()
            scat(3, 3)
            e1.wait()
            gath(5, 1)

            def steady(jo, _2):
                j = jo * 4
                for b in range(4):
                    pltpu.make_async_copy(p_hbm.at[sidx_v.at[j + b]],
                                          rows[b], gsem[b]).wait()
                    scat(j + b, b)
                    bn = (b + 2) % 4
                    pltpu.make_async_copy(rows[bn],
                                          acc_sh.at[didx_v.at[j + b]],
                                          ssem[bn]).wait()
                    pltpu.async_copy(p_hbm.at[sidx_v.at[j + b + 2]],
                                     rows[bn], gsem[bn])
                return 0
            lax.fori_loop(1, 12, steady, 0)

            for j, b in ((48, 0), (49, 1)):
                pltpu.make_async_copy(p_hbm.at[sidx_v.at[j]], rows[b],
                                      gsem[b]).wait()
                scat(j, b)
            for j, b in ((46, 2), (47, 3), (48, 0), (49, 1)):
                pltpu.make_async_copy(rows[b],
                                      acc_sh.at[didx_v.at[j]],
                                      ssem[b]).wait()
            return 0
        lax.fori_loop(0, _AGRP, group, 0)

        plsc.subcore_barrier()
        for t in range(_RPTH // _WB):
            pltpu.sync_copy(
                acc_sh.at[pl.ds(s * _RPTH + t * _WB, _WB)], buf_v)
            pltpu.sync_copy(
                buf_v, out_hbm.at[c, pl.ds(s * _RPTH + t * _WB, _WB)])

    return k(p, srcA, dstA)


def kernel(x, edge_index, batch, W_sem, b_sem, emb, gamma, beta, W1, b1, W2,
           b2, Wc1, bc1, Wc2, bc2):
    sem_feat = x[:, :_BERT]
    sidx = x[:, _BERT:]
    src = edge_index[0]
    dst = edge_index[1]
    dst4 = dst.reshape(_NC * _NT, _DGRP, _GRP, _CH)
    srcA = src.reshape(_NT, _AGRP, _GRP, _CH)
    dstA = dst.reshape(_NT, _AGRP, _GRP, _CH)

    cnts = _deg_sc(dst4)
    cnt0 = cnts[0].reshape(_NPAD, 1)
    cnt1 = cnts[1].reshape(_NPAD, 1)

    h0 = _embed(sem_feat, sidx, W_sem, b_sem.reshape(1, _D), emb,
                gamma.reshape(1, _D), beta.reshape(1, _D))

    p1 = _prep(h0, W1, cnt0, cnt1)
    agg1 = _agg_sc(p1, srcA, dstA).reshape(_NPAD, _D)
    h1, p2 = _mid(agg1, p1, cnt0, cnt1, b1.reshape(1, _D), W2)
    agg2 = _agg_sc(p2, srcA, dstA).reshape(_NPAD, _D)

    wc2p = jnp.concatenate([Wc2, jnp.zeros((_D, _D - 2), _F32)], axis=1)
    bc2p = jnp.concatenate([bc2, jnp.zeros((_D - 2,), _F32)]).reshape(1, _D)
    outp = _final(h1, agg2, p2, cnt0, cnt1,
                  b2.reshape(1, _D), batch.reshape(_N, 1), Wc1,
                  bc1.reshape(1, _D), wc2p, bc2p)
    return outp[:, :2]


# speedup vs baseline: 25.5725x; 1.5641x over previous
import functools

import jax
import jax.numpy as jnp
from jax import lax
from jax.experimental import pallas as pl
from jax.experimental.pallas import tpu as pltpu
from jax.experimental.pallas import tpu_sc as plsc

_N = 10000
_E = 640000
_D = 128
_B = 64
_BERT = 768
_NC = 2
_NT = 16
_DH = _D // _NC
_NPAD = 10240
_RPT = _NPAD // _NT
_EPW = _E // (_NC * _NT)
_CH = 80
_NCH = _EPW // _CH
_NHALF = _NPAD // _NC
_RPTH = _NHALF // _NT
_ECH = _E // (_NT * _CH)
_GRP = 50
_AGRP = _ECH // _GRP
_DGRP = _NCH // _GRP
_WB = 64
_DWB = 2048
_BLK = 1000
_F32 = jnp.float32


def _gelu(x):
    return 0.5 * x * (1.0 + lax.erf(x * 0.7071067811865476))


def _embed_body(sem_ref, sidx_ref, wsem_ref, bsem_ref, emb_ref, gamma_ref,
                beta_ref, o_ref):
    h = jnp.dot(sem_ref[...], wsem_ref[...], preferred_element_type=_F32)
    h = h + bsem_ref[...]
    si = sidx_ref[...].astype(jnp.int32)
    h = h + jnp.where(si <= 0, emb_ref[0:1, :], emb_ref[1:2, :])
    m = jnp.mean(h, axis=-1, keepdims=True)
    v = jnp.mean((h - m) * (h - m), axis=-1, keepdims=True)
    h = (h - m) * lax.rsqrt(v + 1e-5) * gamma_ref[...] + beta_ref[...]
    o_ref[...] = _gelu(h)


def _embed(sem, sidx, w_sem, b_sem, emb, gamma, beta):
    grid = (_N // _BLK,)
    return pl.pallas_call(
        _embed_body,
        grid=grid,
        in_specs=[
            pl.BlockSpec((_BLK, _BERT), lambda i: (i, 0)),
            pl.BlockSpec((_BLK, 1), lambda i: (i, 0)),
            pl.BlockSpec((_BERT, _D), lambda i: (0, 0)),
            pl.BlockSpec((1, _D), lambda i: (0, 0)),
            pl.BlockSpec((2, _D), lambda i: (0, 0)),
            pl.BlockSpec((1, _D), lambda i: (0, 0)),
            pl.BlockSpec((1, _D), lambda i: (0, 0)),
        ],
        out_specs=pl.BlockSpec((_BLK, _D), lambda i: (i, 0)),
        out_shape=jax.ShapeDtypeStruct((_N, _D), _F32),
    )(sem, sidx, w_sem, b_sem, emb, gamma, beta)


def _prep_body(h_ref, w_ref, cnt0_ref, cnt1_ref, pa_ref, pb_ref):
    g = lax.rsqrt(cnt0_ref[...] + cnt1_ref[...] + 1.0)
    p = jnp.dot(h_ref[...], w_ref[...], preferred_element_type=_F32) * g
    pa_ref[...] = p[:, :_DH]
    pb_ref[...] = p[:, _DH:]


def _prep(h, w, cnt0, cnt1):
    grid = (_N // _BLK,)
    half = pl.BlockSpec((_BLK, _DH), lambda i: (i, 0))
    return pl.pallas_call(
        _prep_body,
        grid=grid,
        in_specs=[
            pl.BlockSpec((_BLK, _D), lambda i: (i, 0)),
            pl.BlockSpec((_D, _D), lambda i: (0, 0)),
            pl.BlockSpec((_BLK, 1), lambda i: (i, 0)),
            pl.BlockSpec((_BLK, 1), lambda i: (i, 0)),
        ],
        out_specs=[half, half],
        out_shape=[
            jax.ShapeDtypeStruct((_N, _DH), _F32),
            jax.ShapeDtypeStruct((_N, _DH), _F32),
        ],
    )(h, w, cnt0, cnt1)


def _mid_body(agga_ref, aggb_ref, pa_ref, pb_ref, cnt0_ref, cnt1_ref,
              b1_ref, w2_ref, h1_ref, p2a_ref, p2b_ref):
    g = lax.rsqrt(cnt0_ref[...] + cnt1_ref[...] + 1.0)
    s = jnp.concatenate(
        [agga_ref[...] + pa_ref[...], aggb_ref[...] + pb_ref[...]], axis=1)
    h1 = _gelu(s * g + b1_ref[...])
    h1_ref[...] = h1
    p2 = jnp.dot(h1, w2_ref[...], preferred_element_type=_F32) * g
    p2a_ref[...] = p2[:, :_DH]
    p2b_ref[...] = p2[:, _DH:]


def _mid(agga, aggb, pa, pb, cnt0, cnt1, b1, w2):
    grid = (_N // _BLK,)
    full = pl.BlockSpec((_BLK, _D), lambda i: (i, 0))
    half = pl.BlockSpec((_BLK, _DH), lambda i: (i, 0))
    one = pl.BlockSpec((_BLK, 1), lambda i: (i, 0))
    return pl.pallas_call(
        _mid_body,
        grid=grid,
        in_specs=[
            half, half, half, half, one, one,
            pl.BlockSpec((1, _D), lambda i: (0, 0)),
            pl.BlockSpec((_D, _D), lambda i: (0, 0)),
        ],
        out_specs=[full, half, half],
        out_shape=[
            jax.ShapeDtypeStruct((_N, _D), _F32),
            jax.ShapeDtypeStruct((_N, _DH), _F32),
            jax.ShapeDtypeStruct((_N, _DH), _F32),
        ],
    )(agga, aggb, pa, pb, cnt0, cnt1, b1, w2)


def _final_body(h1_ref, agga_ref, aggb_ref, pa_ref, pb_ref, cnt0_ref,
                cnt1_ref, b2_ref, batch_ref, wc1_ref, bc1_ref, wc2_ref,
                bc2_ref, o_ref, sums_scr, counts_scr):
    i = pl.program_id(0)

    @pl.when(i == 0)
    def _():
        sums_scr[...] = jnp.zeros_like(sums_scr)
        counts_scr[...] = jnp.zeros_like(counts_scr)

    g = lax.rsqrt(cnt0_ref[...] + cnt1_ref[...] + 1.0)
    s = jnp.concatenate(
        [agga_ref[...] + pa_ref[...], aggb_ref[...] + pb_ref[...]], axis=1)
    h2 = h1_ref[...] + _gelu(s * g + b2_ref[...])
    onehot = (batch_ref[...] ==
              lax.broadcasted_iota(jnp.int32, (_BLK, _B), 1)).astype(_F32)
    dn = (((0,), (0,)), ((), ()))
    sums_scr[...] += lax.dot_general(onehot, h2, dn,
                                     preferred_element_type=_F32)
    counts_scr[...] += lax.dot_general(onehot, jnp.ones((_BLK, 1), _F32), dn,
                                       preferred_element_type=_F32)

    @pl.when(i == _N // _BLK - 1)
    def _():
        hg = sums_scr[...] / jnp.maximum(counts_scr[...], 1.0)
        z = _gelu(jnp.dot(hg, wc1_ref[...], preferred_element_type=_F32)
                  + bc1_ref[...])
        o_ref[...] = (jnp.dot(z, wc2_ref[...], preferred_element_type=_F32)
                      + bc2_ref[...])


def _final(h1, agga, aggb, pa, pb, cnt0, cnt1, b2, batch, wc1, bc1, wc2p,
           bc2p):
    grid = (_N // _BLK,)
    full = pl.BlockSpec((_BLK, _D), lambda i: (i, 0))
    half = pl.BlockSpec((_BLK, _DH), lambda i: (i, 0))
    one = pl.BlockSpec((_BLK, 1), lambda i: (i, 0))
    wfull = pl.BlockSpec((_D, _D), lambda i: (0, 0))
    brow = pl.BlockSpec((1, _D), lambda i: (0, 0))
    return pl.pallas_call(
        _final_body,
        grid=grid,
        in_specs=[full, half, half, half, half, one, one, brow,
                  pl.BlockSpec((_BLK, 1), lambda i: (i, 0)),
                  wfull, brow, wfull, brow],
        out_specs=pl.BlockSpec((_B, _D), lambda i: (0, 0)),
        out_shape=jax.ShapeDtypeStruct((_B, _D), _F32),
        scratch_shapes=[
            pltpu.VMEM((_B, _D), _F32),
            pltpu.VMEM((_B, 1), _F32),
        ],
        compiler_params=pltpu.CompilerParams(
            dimension_semantics=("arbitrary",)),
    )(h1, agga, aggb, pa, pb, cnt0, cnt1, b2, batch, wc1, bc1, wc2p, bc2p)


def _deg_sc(dst4):
    mesh = plsc.VectorSubcoreMesh(core_axis_name="c", subcore_axis_name="s",
                                  num_cores=_NC, num_subcores=_NT)

    @functools.partial(
        pl.kernel,
        out_type=jax.ShapeDtypeStruct((_NC, _NPAD), _F32),
        mesh=mesh,
        scratch_types=[
            pltpu.VMEM((_GRP, _CH), jnp.int32),
            pltpu.VMEM((_CH,), _F32),
            pltpu.VMEM((_DWB,), _F32),
            pltpu.VMEM_SHARED((_NPAD,), _F32),
            pltpu.SemaphoreType.DMA,
        ],
    )
    def k(dst_hbm, cnt_hbm, didx_v, ones_v, buf_v, acc_sh, dsem):
        c = lax.axis_index("c")
        s = lax.axis_index("s")
        wid = c * _NT + s

        for kk in range(_CH // 16):
            ones_v[pl.ds(kk * 16, 16)] = jnp.ones((16,), _F32)

        @pl.when(s == 0)
        def _():
            def fill_zero(i, _):
                buf_v[pl.ds(i * 16, 16)] = jnp.zeros((16,), _F32)
                return 0
            lax.fori_loop(0, _DWB // 16, fill_zero, 0)
            for t in range(_NPAD // _DWB):
                pltpu.sync_copy(buf_v, acc_sh.at[pl.ds(t * _DWB, _DWB)])

        plsc.subcore_barrier()

        def group(gi, _):
            pltpu.sync_copy(dst_hbm.at[wid, gi], didx_v)
            for sb in range(_GRP // 10):
                ds = [pltpu.async_copy(
                    ones_v, acc_sh.at[didx_v.at[sb * 10 + t]], dsem,
                    add=True) for t in range(10)]
                for d in ds:
                    d.wait()
            return 0
        lax.fori_loop(0, _DGRP, group, 0)

        plsc.subcore_barrier()

        @pl.when(s == 0)
        def _():
            for t in range(_NPAD // _DWB):
                pltpu.sync_copy(acc_sh.at[pl.ds(t * _DWB, _DWB)], buf_v)
                pltpu.sync_copy(buf_v, cnt_hbm.at[c, pl.ds(t * _DWB, _DWB)])

    return k(dst4)


def _agg_sc(pa, pb, srcA, dstA):
    mesh = plsc.VectorSubcoreMesh(core_axis_name="c", subcore_axis_name="s",
                                  num_cores=_NC, num_subcores=_NT)

    @functools.partial(
        pl.kernel,
        out_type=jax.ShapeDtypeStruct((_NC, _NPAD, _DH), _F32),
        mesh=mesh,
        scratch_types=[
            pltpu.VMEM((_GRP, _CH), jnp.int32),
            pltpu.VMEM((_GRP, _CH), jnp.int32),
            pltpu.VMEM((_CH, _DH), _F32),
            pltpu.VMEM((_CH, _DH), _F32),
            pltpu.VMEM((_CH, _DH), _F32),
            pltpu.VMEM((_CH, _DH), _F32),
            pltpu.VMEM((_WB, _DH), _F32),
            pltpu.VMEM_SHARED((_NPAD, _DH), _F32),
            pltpu.SemaphoreType.DMA,
            pltpu.SemaphoreType.DMA,
            pltpu.SemaphoreType.DMA,
            pltpu.SemaphoreType.DMA,
            pltpu.SemaphoreType.DMA,
            pltpu.SemaphoreType.DMA,
            pltpu.SemaphoreType.DMA,
            pltpu.SemaphoreType.DMA,
        ],
        compiler_params=pltpu.CompilerParams(use_tc_tiling_on_sc=False),
    )
    def k(pa_hbm, pb_hbm, src_hbm, dst_hbm, out_hbm,
          sidx_v, didx_v, r0, r1, r2, r3, buf_v, acc_sh,
          g0, g1, g2, g3, s0, s1, s2, s3):
        c = lax.axis_index("c")
        s = lax.axis_index("s")
        rows = (r0, r1, r2, r3)
        gsem = (g0, g1, g2, g3)
        ssem = (s0, s1, s2, s3)

        def fill_zero(i, _):
            for jj in range(_DH // 16):
                buf_v[i, pl.ds(jj * 16, 16)] = jnp.zeros((16,), _F32)
            return 0
        lax.fori_loop(0, _WB, fill_zero, 0)

        for t in range(_RPT // _WB):
            pltpu.sync_copy(
                buf_v, acc_sh.at[pl.ds(s * _RPT + t * _WB, _WB)])

        plsc.subcore_barrier()

        def gath(j, b):
            @pl.when(c == 0)
            def _():
                pltpu.async_copy(pa_hbm.at[sidx_v.at[j]], rows[b], gsem[b])

            @pl.when(c == 1)
            def _():
                pltpu.async_copy(pb_hbm.at[sidx_v.at[j]], rows[b], gsem[b])

        def gwait(j, b):
            pltpu.make_async_copy(pa_hbm.at[sidx_v.at[j]], rows[b],
                                  gsem[b]).wait()

        def scat(j, b):
            pltpu.async_copy(rows[b], acc_sh.at[didx_v.at[j]],
                             ssem[b], add=True)

        def swait(j, b):
            pltpu.make_async_copy(rows[b], acc_sh.at[didx_v.at[j]],
                                  ssem[b]).wait()

        def group(gi, _):
            pltpu.sync_copy(src_hbm.at[s, gi], sidx_v)
            pltpu.sync_copy(dst_hbm.at[s, gi], didx_v)

            gath(0, 0)
            gath(1, 1)
            gwait(0, 0)
            scat(0, 0)
            gath(2, 2)
            gwait(1, 1)
            scat(1, 1)
            gath(3, 3)
            gwait(2, 2)
            scat(2, 2)
            swait(0, 0)
            gath(4, 0)
            gwait(3, 3)
            scat(3, 3)
            swait(1, 1)
            gath(5, 1)

            def steady(jo, _2):
                j = jo * 4
                for b in range(4):
                    gwait(j + b, b)
                    scat(j + b, b)
                    bn = (b + 2) % 4
                    swait(j + b - 2, bn)
                    gath(j + b + 2, bn)
                return 0
            lax.fori_loop(1, 12, steady, 0)

            for j, b in ((48, 0), (49, 1)):
                gwait(j, b)
                scat(j, b)
            for j, b in ((46, 2), (47, 3), (48, 0), (49, 1)):
                swait(j, b)
            return 0
        lax.fori_loop(0, _AGRP, group, 0)

        plsc.subcore_barrier()
        for t in range(_RPT // _WB):
            pltpu.sync_copy(
                acc_sh.at[pl.ds(s * _RPT + t * _WB, _WB)], buf_v)
            pltpu.sync_copy(
                buf_v, out_hbm.at[c, pl.ds(s * _RPT + t * _WB, _WB)])

    return k(pa, pb, srcA, dstA)


def kernel(x, edge_index, batch, W_sem, b_sem, emb, gamma, beta, W1, b1, W2,
           b2, Wc1, bc1, Wc2, bc2):
    sem_feat = x[:, :_BERT]
    sidx = x[:, _BERT:]
    src = edge_index[0]
    dst = edge_index[1]
    dst4 = dst.reshape(_NC * _NT, _DGRP, _GRP, _CH)
    srcA = src.reshape(_NT, _AGRP, _GRP, _CH)
    dstA = dst.reshape(_NT, _AGRP, _GRP, _CH)

    cnts = _deg_sc(dst4)
    cnt0 = cnts[0].reshape(_NPAD, 1)
    cnt1 = cnts[1].reshape(_NPAD, 1)

    h0 = _embed(sem_feat, sidx, W_sem, b_sem.reshape(1, _D), emb,
                gamma.reshape(1, _D), beta.reshape(1, _D))

    p1a, p1b = _prep(h0, W1, cnt0, cnt1)
    aggs1 = _agg_sc(p1a, p1b, srcA, dstA)
    h1, p2a, p2b = _mid(aggs1[0], aggs1[1], p1a, p1b, cnt0, cnt1,
                        b1.reshape(1, _D), W2)
    aggs2 = _agg_sc(p2a, p2b, srcA, dstA)

    wc2p = jnp.concatenate([Wc2, jnp.zeros((_D, _D - 2), _F32)], axis=1)
    bc2p = jnp.concatenate([bc2, jnp.zeros((_D - 2,), _F32)]).reshape(1, _D)
    outp = _final(h1, aggs2[0], aggs2[1], p2a, p2b, cnt0, cnt1,
                  b2.reshape(1, _D), batch.reshape(_N, 1), Wc1,
                  bc1.reshape(1, _D), wc2p, bc2p)
    return outp[:, :2]


# speedup vs baseline: 25.5890x; 1.0006x over previous
import functools

import jax
import jax.numpy as jnp
from jax import lax
from jax.experimental import pallas as pl
from jax.experimental.pallas import tpu as pltpu
from jax.experimental.pallas import tpu_sc as plsc

_N = 10000
_E = 640000
_D = 128
_B = 64
_BERT = 768
_NC = 2
_NT = 16
_DH = _D // _NC
_NPAD = 10240
_RPT = _NPAD // _NT
_EPW = _E // (_NC * _NT)
_CH = 80
_NCH = _EPW // _CH
_NHALF = _NPAD // _NC
_RPTH = _NHALF // _NT
_ECH = _E // (_NT * _CH)
_GRP = 50
_AGRP = _ECH // _GRP
_DGRP = _NCH // _GRP
_WB = 64
_DWB = 2048
_BLK = 1000
_F32 = jnp.float32


def _gelu(x):
    return 0.5 * x * (1.0 + lax.erf(x * 0.7071067811865476))


def _embed_body(sem_ref, sidx_ref, wsem_ref, bsem_ref, emb_ref, gamma_ref,
                beta_ref, o_ref):
    h = jnp.dot(sem_ref[...], wsem_ref[...], preferred_element_type=_F32)
    h = h + bsem_ref[...]
    si = sidx_ref[...].astype(jnp.int32)
    h = h + jnp.where(si <= 0, emb_ref[0:1, :], emb_ref[1:2, :])
    m = jnp.mean(h, axis=-1, keepdims=True)
    v = jnp.mean((h - m) * (h - m), axis=-1, keepdims=True)
    h = (h - m) * lax.rsqrt(v + 1e-5) * gamma_ref[...] + beta_ref[...]
    o_ref[...] = _gelu(h)


def _embed(sem, sidx, w_sem, b_sem, emb, gamma, beta):
    grid = (_N // _BLK,)
    return pl.pallas_call(
        _embed_body,
        grid=grid,
        in_specs=[
            pl.BlockSpec((_BLK, _BERT), lambda i: (i, 0)),
            pl.BlockSpec((_BLK, 1), lambda i: (i, 0)),
            pl.BlockSpec((_BERT, _D), lambda i: (0, 0)),
            pl.BlockSpec((1, _D), lambda i: (0, 0)),
            pl.BlockSpec((2, _D), lambda i: (0, 0)),
            pl.BlockSpec((1, _D), lambda i: (0, 0)),
            pl.BlockSpec((1, _D), lambda i: (0, 0)),
        ],
        out_specs=pl.BlockSpec((_BLK, _D), lambda i: (i, 0)),
        out_shape=jax.ShapeDtypeStruct((_N, _D), _F32),
    )(sem, sidx, w_sem, b_sem, emb, gamma, beta)


def _prep_body(h_ref, w_ref, cnt0_ref, cnt1_ref, pa_ref, pb_ref):
    g = lax.rsqrt(cnt0_ref[...] + cnt1_ref[...] + 1.0)
    p = jnp.dot(h_ref[...], w_ref[...], preferred_element_type=_F32) * g
    pa_ref[...] = p[:, :_DH]
    pb_ref[...] = p[:, _DH:]


def _prep(h, w, cnt0, cnt1):
    grid = (_N // _BLK,)
    half = pl.BlockSpec((_BLK, _DH), lambda i: (i, 0))
    return pl.pallas_call(
        _prep_body,
        grid=grid,
        in_specs=[
            pl.BlockSpec((_BLK, _D), lambda i: (i, 0)),
            pl.BlockSpec((_D, _D), lambda i: (0, 0)),
            pl.BlockSpec((_BLK, 1), lambda i: (i, 0)),
            pl.BlockSpec((_BLK, 1), lambda i: (i, 0)),
        ],
        out_specs=[half, half],
        out_shape=[
            jax.ShapeDtypeStruct((_N, _DH), _F32),
            jax.ShapeDtypeStruct((_N, _DH), _F32),
        ],
    )(h, w, cnt0, cnt1)


def _mid_body(agga_ref, aggb_ref, pa_ref, pb_ref, cnt0_ref, cnt1_ref,
              b1_ref, w2_ref, h1_ref, p2a_ref, p2b_ref):
    g = lax.rsqrt(cnt0_ref[...] + cnt1_ref[...] + 1.0)
    s = jnp.concatenate(
        [agga_ref[...] + pa_ref[...], aggb_ref[...] + pb_ref[...]], axis=1)
    h1 = _gelu(s * g + b1_ref[...])
    h1_ref[...] = h1
    p2 = jnp.dot(h1, w2_ref[...], preferred_element_type=_F32) * g
    p2a_ref[...] = p2[:, :_DH]
    p2b_ref[...] = p2[:, _DH:]


def _mid(agga, aggb, pa, pb, cnt0, cnt1, b1, w2):
    grid = (_N // _BLK,)
    full = pl.BlockSpec((_BLK, _D), lambda i: (i, 0))
    half = pl.BlockSpec((_BLK, _DH), lambda i: (i, 0))
    one = pl.BlockSpec((_BLK, 1), lambda i: (i, 0))
    return pl.pallas_call(
        _mid_body,
        grid=grid,
        in_specs=[
            half, half, half, half, one, one,
            pl.BlockSpec((1, _D), lambda i: (0, 0)),
            pl.BlockSpec((_D, _D), lambda i: (0, 0)),
        ],
        out_specs=[full, half, half],
        out_shape=[
            jax.ShapeDtypeStruct((_N, _D), _F32),
            jax.ShapeDtypeStruct((_N, _DH), _F32),
            jax.ShapeDtypeStruct((_N, _DH), _F32),
        ],
    )(agga, aggb, pa, pb, cnt0, cnt1, b1, w2)


def _final_body(h1_ref, agga_ref, aggb_ref, pa_ref, pb_ref, cnt0_ref,
                cnt1_ref, b2_ref, batch_ref, wc1_ref, bc1_ref, wc2_ref,
                bc2_ref, o_ref, sums_scr, counts_scr):
    i = pl.program_id(0)

    @pl.when(i == 0)
    def _():
        sums_scr[...] = jnp.zeros_like(sums_scr)
        counts_scr[...] = jnp.zeros_like(counts_scr)

    g = lax.rsqrt(cnt0_ref[...] + cnt1_ref[...] + 1.0)
    s = jnp.concatenate(
        [agga_ref[...] + pa_ref[...], aggb_ref[...] + pb_ref[...]], axis=1)
    h2 = h1_ref[...] + _gelu(s * g + b2_ref[...])
    onehot = (batch_ref[...] ==
              lax.broadcasted_iota(jnp.int32, (_BLK, _B), 1)).astype(_F32)
    dn = (((0,), (0,)), ((), ()))
    sums_scr[...] += lax.dot_general(onehot, h2, dn,
                                     preferred_element_type=_F32)
    counts_scr[...] += lax.dot_general(onehot, jnp.ones((_BLK, 1), _F32), dn,
                                       preferred_element_type=_F32)

    @pl.when(i == _N // _BLK - 1)
    def _():
        hg = sums_scr[...] / jnp.maximum(counts_scr[...], 1.0)
        z = _gelu(jnp.dot(hg, wc1_ref[...], preferred_element_type=_F32)
                  + bc1_ref[...])
        o_ref[...] = (jnp.dot(z, wc2_ref[...], preferred_element_type=_F32)
                      + bc2_ref[...])


def _final(h1, agga, aggb, pa, pb, cnt0, cnt1, b2, batch, wc1, bc1, wc2p,
           bc2p):
    grid = (_N // _BLK,)
    full = pl.BlockSpec((_BLK, _D), lambda i: (i, 0))
    half = pl.BlockSpec((_BLK, _DH), lambda i: (i, 0))
    one = pl.BlockSpec((_BLK, 1), lambda i: (i, 0))
    wfull = pl.BlockSpec((_D, _D), lambda i: (0, 0))
    brow = pl.BlockSpec((1, _D), lambda i: (0, 0))
    return pl.pallas_call(
        _final_body,
        grid=grid,
        in_specs=[full, half, half, half, half, one, one, brow,
                  pl.BlockSpec((_BLK, 1), lambda i: (i, 0)),
                  wfull, brow, wfull, brow],
        out_specs=pl.BlockSpec((_B, _D), lambda i: (0, 0)),
        out_shape=jax.ShapeDtypeStruct((_B, _D), _F32),
        scratch_shapes=[
            pltpu.VMEM((_B, _D), _F32),
            pltpu.VMEM((_B, 1), _F32),
        ],
        compiler_params=pltpu.CompilerParams(
            dimension_semantics=("arbitrary",)),
    )(h1, agga, aggb, pa, pb, cnt0, cnt1, b2, batch, wc1, bc1, wc2p, bc2p)


def _deg_sc(dst4):
    mesh = plsc.VectorSubcoreMesh(core_axis_name="c", subcore_axis_name="s",
                                  num_cores=_NC, num_subcores=_NT)

    @functools.partial(
        pl.kernel,
        out_type=jax.ShapeDtypeStruct((_NC, _NPAD), _F32),
        mesh=mesh,
        scratch_types=[
            pltpu.VMEM((_GRP, _CH), jnp.int32),
            pltpu.VMEM((_CH,), _F32),
            pltpu.VMEM((_DWB,), _F32),
            pltpu.VMEM_SHARED((_NPAD,), _F32),
            pltpu.SemaphoreType.DMA,
        ],
    )
    def k(dst_hbm, cnt_hbm, didx_v, ones_v, buf_v, acc_sh, dsem):
        c = lax.axis_index("c")
        s = lax.axis_index("s")
        wid = c * _NT + s

        for kk in range(_CH // 16):
            ones_v[pl.ds(kk * 16, 16)] = jnp.ones((16,), _F32)

        @pl.when(s == 0)
        def _():
            def fill_zero(i, _):
                buf_v[pl.ds(i * 16, 16)] = jnp.zeros((16,), _F32)
                return 0
            lax.fori_loop(0, _DWB // 16, fill_zero, 0)
            for t in range(_NPAD // _DWB):
                pltpu.sync_copy(buf_v, acc_sh.at[pl.ds(t * _DWB, _DWB)])

        plsc.subcore_barrier()

        def group(gi, _):
            pltpu.sync_copy(dst_hbm.at[wid, gi], didx_v)
            for sb in range(_GRP // 10):
                ds = [pltpu.async_copy(
                    ones_v, acc_sh.at[didx_v.at[sb * 10 + t]], dsem,
                    add=True) for t in range(10)]
                for d in ds:
                    d.wait()
            return 0
        lax.fori_loop(0, _DGRP, group, 0)

        plsc.subcore_barrier()

        @pl.when(s == 0)
        def _():
            for t in range(_NPAD // _DWB):
                pltpu.sync_copy(acc_sh.at[pl.ds(t * _DWB, _DWB)], buf_v)
                pltpu.sync_copy(buf_v, cnt_hbm.at[c, pl.ds(t * _DWB, _DWB)])

    return k(dst4)


def _agg_sc(pa, pb, srcA, dstA):
    mesh = plsc.VectorSubcoreMesh(core_axis_name="c", subcore_axis_name="s",
                                  num_cores=_NC, num_subcores=_NT)

    @functools.partial(
        pl.kernel,
        out_type=jax.ShapeDtypeStruct((_NC, _NPAD, _DH), _F32),
        mesh=mesh,
        scratch_types=[
            pltpu.VMEM((_GRP, _CH), jnp.int32),
            pltpu.VMEM((_GRP, _CH), jnp.int32),
            pltpu.VMEM((_CH, _DH), _F32),
            pltpu.VMEM((_CH, _DH), _F32),
            pltpu.VMEM((_CH, _DH), _F32),
            pltpu.VMEM((_CH, _DH), _F32),
            pltpu.VMEM((_CH, _DH), _F32),
            pltpu.VMEM((_WB, _DH), _F32),
            pltpu.VMEM_SHARED((_NPAD, _DH), _F32),
            pltpu.SemaphoreType.DMA,
            pltpu.SemaphoreType.DMA,
            pltpu.SemaphoreType.DMA,
            pltpu.SemaphoreType.DMA,
            pltpu.SemaphoreType.DMA,
            pltpu.SemaphoreType.DMA,
            pltpu.SemaphoreType.DMA,
            pltpu.SemaphoreType.DMA,
            pltpu.SemaphoreType.DMA,
            pltpu.SemaphoreType.DMA,
        ],
        compiler_params=pltpu.CompilerParams(use_tc_tiling_on_sc=False),
    )
    def k(pa_hbm, pb_hbm, src_hbm, dst_hbm, out_hbm,
          sidx_v, didx_v, r0, r1, r2, r3, r4, buf_v, acc_sh,
          g0, g1, g2, g3, g4, s0, s1, s2, s3, s4):
        c = lax.axis_index("c")
        s = lax.axis_index("s")
        rows = (r0, r1, r2, r3, r4)
        gsem = (g0, g1, g2, g3, g4)
        ssem = (s0, s1, s2, s3, s4)

        def fill_zero(i, _):
            for jj in range(_DH // 16):
                buf_v[i, pl.ds(jj * 16, 16)] = jnp.zeros((16,), _F32)
            return 0
        lax.fori_loop(0, _WB, fill_zero, 0)

        for t in range(_RPT // _WB):
            pltpu.sync_copy(
                buf_v, acc_sh.at[pl.ds(s * _RPT + t * _WB, _WB)])

        plsc.subcore_barrier()

        def gath(j, b):
            @pl.when(c == 0)
            def _():
                pltpu.async_copy(pa_hbm.at[sidx_v.at[j]], rows[b], gsem[b])

            @pl.when(c == 1)
            def _():
                pltpu.async_copy(pb_hbm.at[sidx_v.at[j]], rows[b], gsem[b])

        def gwait(j, b):
            pltpu.make_async_copy(pa_hbm.at[sidx_v.at[j]], rows[b],
                                  gsem[b]).wait()

        def scat(j, b):
            pltpu.async_copy(rows[b], acc_sh.at[didx_v.at[j]],
                             ssem[b], add=True)

        def swait(j, b):
            pltpu.make_async_copy(rows[b], acc_sh.at[didx_v.at[j]],
                                  ssem[b]).wait()

        def group(gi, _):
            pltpu.sync_copy(src_hbm.at[s, gi], sidx_v)
            pltpu.sync_copy(dst_hbm.at[s, gi], didx_v)

            gath(0, 0)
            gath(1, 1)
            gwait(0, 0)
            scat(0, 0)
            gath(2, 2)
            gwait(1, 1)
            scat(1, 1)
            gath(3, 3)
            gwait(2, 2)
            scat(2, 2)
            gath(4, 4)
            gwait(3, 3)
            scat(3, 3)
            swait(0, 0)
            gath(5, 0)
            gwait(4, 4)
            scat(4, 4)
            swait(1, 1)
            gath(6, 1)

            def steady(jo, _2):
                j = jo * 5
                for b in range(5):
                    gwait(j + b, b)
                    scat(j + b, b)
                    bn = (b + 2) % 5
                    swait(j + b - 3, bn)
                    gath(j + b + 2, bn)
                return 0
            lax.fori_loop(1, 9, steady, 0)

            gwait(45, 0)
            scat(45, 0)
            swait(42, 2)
            gath(47, 2)
            gwait(46, 1)
            scat(46, 1)
            swait(43, 3)
            gath(48, 3)
            gwait(47, 2)
            scat(47, 2)
            swait(44, 4)
            gath(49, 4)
            gwait(48, 3)
            scat(48, 3)
            gwait(49, 4)
            scat(49, 4)
            for j, b in ((45, 0), (46, 1), (47, 2), (48, 3), (49, 4)):
                swait(j, b)
            return 0
        lax.fori_loop(0, _AGRP, group, 0)

        plsc.subcore_barrier()
        for t in range(_RPT // _WB):
            pltpu.sync_copy(
                acc_sh.at[pl.ds(s * _RPT + t * _WB, _WB)], buf_v)
            pltpu.sync_copy(
                buf_v, out_hbm.at[c, pl.ds(s * _RPT + t * _WB, _WB)])

    return k(pa, pb, srcA, dstA)


def kernel(x, edge_index, batch, W_sem, b_sem, emb, gamma, beta, W1, b1, W2,
           b2, Wc1, bc1, Wc2, bc2):
    sem_feat = x[:, :_BERT]
    sidx = x[:, _BERT:]
    src = edge_index[0]
    dst = edge_index[1]
    dst4 = dst.reshape(_NC * _NT, _DGRP, _GRP, _CH)
    srcA = src.reshape(_NT, _AGRP, _GRP, _CH)
    dstA = dst.reshape(_NT, _AGRP, _GRP, _CH)

    cnts = _deg_sc(dst4)
    cnt0 = cnts[0].reshape(_NPAD, 1)
    cnt1 = cnts[1].reshape(_NPAD, 1)

    h0 = _embed(sem_feat, sidx, W_sem, b_sem.reshape(1, _D), emb,
                gamma.reshape(1, _D), beta.reshape(1, _D))

    p1a, p1b = _prep(h0, W1, cnt0, cnt1)
    aggs1 = _agg_sc(p1a, p1b, srcA, dstA)
    h1, p2a, p2b = _mid(aggs1[0], aggs1[1], p1a, p1b, cnt0, cnt1,
                        b1.reshape(1, _D), W2)
    aggs2 = _agg_sc(p2a, p2b, srcA, dstA)

    wc2p = jnp.concatenate([Wc2, jnp.zeros((_D, _D - 2), _F32)], axis=1)
    bc2p = jnp.concatenate([bc2, jnp.zeros((_D - 2,), _F32)]).reshape(1, _D)
    outp = _final(h1, aggs2[0], aggs2[1], p2a, p2b, cnt0, cnt1,
                  b2.reshape(1, _D), batch.reshape(_N, 1), Wc1,
                  bc1.reshape(1, _D), wc2p, bc2p)
    return outp[:, :2]
